# baseline (speedup 1.0000x reference)
import functools

import jax
import jax.numpy as jnp
from jax import lax
from jax.experimental import pallas as pl
from jax.experimental.pallas import tpu as pltpu
from jax.experimental.pallas import tpu_sc as plsc

_B = 16384
_D = 128
_NW = 32
_NSPLIT = 2
_BS = _B // _NSPLIT
_BPW = _BS // _NW
_CH = 256
_NCH = _BPW // _CH
_BBLK = 1024


def _gather_body(split, uidx_hbm, iidx_hbm, t_umf, t_imf, t_umlp, t_imlp,
                 o_x, o_umf, o_imf,
                 uidx_v, iidx_v, buf0, buf1, gsem0, gsem1, ssem0, ssem1):
    wid = lax.axis_index("s") * 2 + lax.axis_index("c")
    base = wid * _BPW
    src = split * _BS + wid * _BPW
    pltpu.sync_copy(uidx_hbm.at[pl.ds(src, _BPW)], uidx_v)
    pltpu.sync_copy(iidx_hbm.at[pl.ds(src, _BPW)], iidx_v)
    bufs = (buf0, buf1)
    gsems = (gsem0, gsem1)
    ssems = (ssem0, ssem1)
    tasks = []
    for c in range(_NCH):
        r0 = c * _CH
        for tbl, idxv, out, col in ((t_umlp, uidx_v, o_x, 0),
                                    (t_imlp, iidx_v, o_x, _D),
                                    (t_umf, uidx_v, o_umf, None),
                                    (t_imf, iidx_v, o_imf, None)):
            tasks.append((tbl, idxv, r0, out, col))
    scat = [None, None]
    for t, (tbl, idxv, r0, out, col) in enumerate(tasks):
        b = t % 2
        if scat[b] is not None:
            scat[b].wait()
        pltpu.async_copy(tbl.at[idxv.at[pl.ds(r0, _CH)]], bufs[b],
                         gsems[b]).wait()
        if col is None:
            dst = out.at[pl.ds(base + r0, _CH)]
        else:
            dst = out.at[pl.ds(base + r0, _CH), pl.ds(col, _D)]
        scat[b] = pltpu.async_copy(bufs[b], dst, ssems[b])
    scat[0].wait()
    scat[1].wait()


def _gather_sc(split, uidx, iidx, t_umf, t_imf, t_umlp, t_imlp):
    mesh = plsc.VectorSubcoreMesh(core_axis_name="c", subcore_axis_name="s")
    run = functools.partial(
        pl.kernel, mesh=mesh,
        out_type=[jax.ShapeDtypeStruct((_BS, 2 * _D), jnp.float32),
                  jax.ShapeDtypeStruct((_BS, _D), jnp.float32),
                  jax.ShapeDtypeStruct((_BS, _D), jnp.float32)],
        scratch_types=[
            pltpu.VMEM((_BPW,), jnp.int32),
            pltpu.VMEM((_BPW,), jnp.int32),
            pltpu.VMEM((_CH, _D), jnp.float32),
            pltpu.VMEM((_CH, _D), jnp.float32),
            pltpu.SemaphoreType.DMA,
            pltpu.SemaphoreType.DMA,
            pltpu.SemaphoreType.DMA,
            pltpu.SemaphoreType.DMA,
        ],
    )(functools.partial(_gather_body, split))
    return run(uidx, iidx, t_umf, t_imf, t_umlp, t_imlp)


def _dense_body(x, u_mf, i_mf, W1, b1, W2, b2, W3, b3, wo, bo, *rest):
    out = rest[-1]
    f32 = jnp.float32
    bf16 = jnp.bfloat16
    h = jnp.dot(x[...].astype(bf16), W1[...], preferred_element_type=f32)
    h = jnp.maximum(h + b1[...], 0.0).astype(bf16)
    h = jnp.dot(h, W2[...], preferred_element_type=f32)
    h = jnp.maximum(h + b2[...], 0.0).astype(bf16)
    h = jnp.dot(h, W3[...], preferred_element_type=f32)
    h = jnp.maximum(h + b3[...], 0.0)
    mf = u_mf[...] * i_mf[...]
    wo_v = wo[...]
    acc = jnp.sum(mf * wo_v[:, :_D], axis=1, keepdims=True)
    acc = acc + jnp.sum(h * wo_v[:, _D:], axis=1, keepdims=True)
    res = jax.nn.sigmoid(acc + bo[...])
    out[...] = res.reshape(out.shape)


def _dense_tc(split, x_g, u_mf_g, i_mf_g, W1, b1, W2, b2, W3, b3, wov, bov,
              prev_out):
    off = split * (_BS // _BBLK)
    blk = lambda r, c: pl.BlockSpec((r, c), lambda i: (0, 0))
    act = pl.BlockSpec((_BBLK, _D), lambda i: (i, 0))
    out_spec = pl.BlockSpec((_BBLK,), lambda i: (i + off,))
    in_specs = [pl.BlockSpec((_BBLK, 2 * _D), lambda i: (i, 0)), act, act,
                blk(256, 1024), blk(1, 1024),
                blk(1024, 512), blk(1, 512),
                blk(512, 128), blk(1, 128),
                blk(1, 256), blk(1, 1)]
    args = [x_g, u_mf_g, i_mf_g, W1, b1, W2, b2, W3, b3, wov, bov]
    aliases = {}
    if prev_out is not None:
        in_specs.append(pl.BlockSpec(memory_space=pl.ANY))
        args.append(prev_out)
        aliases = {11: 0}
    return pl.pallas_call(
        _dense_body,
        grid=(_BS // _BBLK,),
        in_specs=in_specs,
        out_specs=out_spec,
        out_shape=jax.ShapeDtypeStruct((_B,), jnp.float32),
        input_output_aliases=aliases,
        compiler_params=pltpu.CompilerParams(
            dimension_semantics=("arbitrary",)),
    )(*args)


def kernel(user_indices, item_indices, emb_user_mf, emb_item_mf,
           emb_user_mlp, emb_item_mlp, W1, b1, W2, b2, W3, b3, W_out, b_out):
    uidx = user_indices.astype(jnp.int32)
    iidx = item_indices.astype(jnp.int32)
    bf16 = jnp.bfloat16
    W1c = W1.astype(bf16)
    W2c = W2.astype(bf16)
    W3c = W3.astype(bf16)
    b1v = b1.reshape(1, -1)
    b2v = b2.reshape(1, -1)
    b3v = b3.reshape(1, -1)
    wov = W_out.reshape(1, -1)
    bov = b_out.reshape(1, 1)
    out = None
    for s in range(_NSPLIT):
        x_g, o_umf, o_imf = _gather_sc(
            s, uidx, iidx, emb_user_mf, emb_item_mf,
            emb_user_mlp, emb_item_mlp)
        out = _dense_tc(s, x_g, o_umf, o_imf,
                        W1c, b1v, W2c, b2v, W3c, b3v, wov, bov, out)
    return out.reshape(_B, 1)


# speedup vs baseline: 1.1643x; 1.1643x over previous
import functools

import jax
import jax.numpy as jnp
from jax import lax
from jax.experimental import pallas as pl
from jax.experimental.pallas import tpu as pltpu
from jax.experimental.pallas import tpu_sc as plsc

_B = 16384
_D = 128
_NW = 32
_NSPLIT = 2
_BS = _B // _NSPLIT
_BPW = _BS // _NW
_CH = 256
_NCH = _BPW // _CH
_BBLK = 1024


def _gather_body(split, uidx_hbm, iidx_hbm, t_umf, t_imf, t_umlp, t_imlp,
                 o_x, o_umf, o_imf,
                 uidx_v, iidx_v, buf0, buf1, gsem0, gsem1, ssem0, ssem1):
    wid = lax.axis_index("s") * 2 + lax.axis_index("c")
    base = wid * _BPW
    src = split * _BS + wid * _BPW
    pltpu.sync_copy(uidx_hbm.at[pl.ds(src, _BPW)], uidx_v)
    pltpu.sync_copy(iidx_hbm.at[pl.ds(src, _BPW)], iidx_v)
    bufs = (buf0, buf1)
    gsems = (gsem0, gsem1)
    ssems = (ssem0, ssem1)
    tasks = []
    for c in range(_NCH):
        r0 = c * _CH
        for tbl, idxv, out, col in ((t_umlp, uidx_v, o_x, 0),
                                    (t_imlp, iidx_v, o_x, _D),
                                    (t_umf, uidx_v, o_umf, None),
                                    (t_imf, iidx_v, o_imf, None)):
            tasks.append((tbl, idxv, r0, out, col))
    scat = [None, None]
    for t, (tbl, idxv, r0, out, col) in enumerate(tasks):
        b = t % 2
        if scat[b] is not None:
            scat[b].wait()
        pltpu.async_copy(tbl.at[idxv.at[pl.ds(r0, _CH)]], bufs[b],
                         gsems[b]).wait()
        if col is None:
            dst = out.at[pl.ds(base + r0, _CH)]
        else:
            dst = out.at[pl.ds(base + r0, _CH), pl.ds(col, _D)]
        scat[b] = pltpu.async_copy(bufs[b], dst, ssems[b])
    scat[0].wait()
    scat[1].wait()


def _gather_sc(split, uidx, iidx, t_umf, t_imf, t_umlp, t_imlp):
    mesh = plsc.VectorSubcoreMesh(core_axis_name="c", subcore_axis_name="s")
    run = functools.partial(
        pl.kernel, mesh=mesh,
        out_type=[jax.ShapeDtypeStruct((_BS, 2 * _D), jnp.float32),
                  jax.ShapeDtypeStruct((_BS, _D), jnp.float32),
                  jax.ShapeDtypeStruct((_BS, _D), jnp.float32)],
        scratch_types=[
            pltpu.VMEM((_BPW,), jnp.int32),
            pltpu.VMEM((_BPW,), jnp.int32),
            pltpu.VMEM((_CH, _D), jnp.float32),
            pltpu.VMEM((_CH, _D), jnp.float32),
            pltpu.SemaphoreType.DMA,
            pltpu.SemaphoreType.DMA,
            pltpu.SemaphoreType.DMA,
            pltpu.SemaphoreType.DMA,
        ],
    )(functools.partial(_gather_body, split))
    return run(uidx, iidx, t_umf, t_imf, t_umlp, t_imlp)


def _dense_body(x, u_mf, i_mf, W1, b1, W2, b2, W3, b3, wo, bo, *rest):
    out = rest[-1]
    f32 = jnp.float32
    bf16 = jnp.bfloat16
    h = jnp.dot(x[...].astype(bf16), W1[...], preferred_element_type=f32)
    h = jnp.maximum(h + b1[...], 0.0).astype(bf16)
    h = jnp.dot(h, W2[...], preferred_element_type=f32)
    h = jnp.maximum(h + b2[...], 0.0).astype(bf16)
    h = jnp.dot(h, W3[...], preferred_element_type=f32)
    h = jnp.maximum(h + b3[...], 0.0)
    mf = u_mf[...] * i_mf[...]
    wo_v = wo[...]
    q = mf * wo_v[:, :_D] + h * wo_v[:, _D:]
    q3 = q.reshape(_BBLK // _D, _D, _D)
    acc = jnp.sum(q3, axis=2)
    out[...] = jax.nn.sigmoid(acc + bo[...])


def _dense_tc(split, x_g, u_mf_g, i_mf_g, W1, b1, W2, b2, W3, b3, wov, bov,
              prev_out):
    off = split * (_BS // _BBLK)
    blk = lambda r, c: pl.BlockSpec((r, c), lambda i: (0, 0))
    act = pl.BlockSpec((_BBLK, _D), lambda i: (i, 0))
    out_spec = pl.BlockSpec((_BBLK // _D, _D), lambda i: (i + off, 0))
    in_specs = [pl.BlockSpec((_BBLK, 2 * _D), lambda i: (i, 0)), act, act,
                blk(256, 1024), blk(1, 1024),
                blk(1024, 512), blk(1, 512),
                blk(512, 128), blk(1, 128),
                blk(1, 256), blk(1, 1)]
    args = [x_g, u_mf_g, i_mf_g, W1, b1, W2, b2, W3, b3, wov, bov]
    aliases = {}
    if prev_out is not None:
        in_specs.append(pl.BlockSpec(memory_space=pl.ANY))
        args.append(prev_out)
        aliases = {11: 0}
    return pl.pallas_call(
        _dense_body,
        grid=(_BS // _BBLK,),
        in_specs=in_specs,
        out_specs=out_spec,
        out_shape=jax.ShapeDtypeStruct((_B // _D, _D), jnp.float32),
        input_output_aliases=aliases,
        compiler_params=pltpu.CompilerParams(
            dimension_semantics=("arbitrary",)),
    )(*args)


def kernel(user_indices, item_indices, emb_user_mf, emb_item_mf,
           emb_user_mlp, emb_item_mlp, W1, b1, W2, b2, W3, b3, W_out, b_out):
    uidx = user_indices.astype(jnp.int32)
    iidx = item_indices.astype(jnp.int32)
    bf16 = jnp.bfloat16
    W1c = W1.astype(bf16)
    W2c = W2.astype(bf16)
    W3c = W3.astype(bf16)
    b1v = b1.reshape(1, -1)
    b2v = b2.reshape(1, -1)
    b3v = b3.reshape(1, -1)
    wov = W_out.reshape(1, -1)
    bov = b_out.reshape(1, 1)
    out = None
    for s in range(_NSPLIT):
        x_g, o_umf, o_imf = _gather_sc(
            s, uidx, iidx, emb_user_mf, emb_item_mf,
            emb_user_mlp, emb_item_mlp)
        out = _dense_tc(s, x_g, o_umf, o_imf,
                        W1c, b1v, W2c, b2v, W3c, b3v, wov, bov, out)
    return out.reshape(_B, 1)


# speedup vs baseline: 1.2362x; 1.0617x over previous
import functools

import jax
import jax.numpy as jnp
from jax import lax
from jax.experimental import pallas as pl
from jax.experimental.pallas import tpu as pltpu
from jax.experimental.pallas import tpu_sc as plsc

_B = 16384
_D = 128
_NW = 32
_NSPLIT = 2
_BS = _B // _NSPLIT
_BPW = _BS // _NW
_CH = 256
_NCH = _BPW // _CH
_BBLK = 2048


def _gather_body(split, uidx_hbm, iidx_hbm, t_umf, t_imf, t_umlp, t_imlp,
                 o_x, o_umf, o_imf,
                 uidx_v, iidx_v, buf0, buf1, gsem0, gsem1, ssem0, ssem1):
    wid = lax.axis_index("s") * 2 + lax.axis_index("c")
    base = wid * _BPW
    src = split * _BS + wid * _BPW
    pltpu.sync_copy(uidx_hbm.at[pl.ds(src, _BPW)], uidx_v)
    pltpu.sync_copy(iidx_hbm.at[pl.ds(src, _BPW)], iidx_v)
    bufs = (buf0, buf1)
    gsems = (gsem0, gsem1)
    ssems = (ssem0, ssem1)
    tasks = []
    for c in range(_NCH):
        r0 = c * _CH
        for tbl, idxv, out, col in ((t_umlp, uidx_v, o_x, 0),
                                    (t_imlp, iidx_v, o_x, _D),
                                    (t_umf, uidx_v, o_umf, None),
                                    (t_imf, iidx_v, o_imf, None)):
            tasks.append((tbl, idxv, r0, out, col))
    scat = [None, None]
    for t, (tbl, idxv, r0, out, col) in enumerate(tasks):
        b = t % 2
        if scat[b] is not None:
            scat[b].wait()
        pltpu.async_copy(tbl.at[idxv.at[pl.ds(r0, _CH)]], bufs[b],
                         gsems[b]).wait()
        if col is None:
            dst = out.at[pl.ds(base + r0, _CH)]
        else:
            dst = out.at[pl.ds(base + r0, _CH), pl.ds(col, _D)]
        scat[b] = pltpu.async_copy(bufs[b], dst, ssems[b])
    scat[0].wait()
    scat[1].wait()


def _gather_sc(split, uidx, iidx, t_umf, t_imf, t_umlp, t_imlp):
    mesh = plsc.VectorSubcoreMesh(core_axis_name="c", subcore_axis_name="s")
    run = functools.partial(
        pl.kernel, mesh=mesh,
        out_type=[jax.ShapeDtypeStruct((_BS, 2 * _D), jnp.float32),
                  jax.ShapeDtypeStruct((_BS, _D), jnp.float32),
                  jax.ShapeDtypeStruct((_BS, _D), jnp.float32)],
        scratch_types=[
            pltpu.VMEM((_BPW,), jnp.int32),
            pltpu.VMEM((_BPW,), jnp.int32),
            pltpu.VMEM((_CH, _D), jnp.float32),
            pltpu.VMEM((_CH, _D), jnp.float32),
            pltpu.SemaphoreType.DMA,
            pltpu.SemaphoreType.DMA,
            pltpu.SemaphoreType.DMA,
            pltpu.SemaphoreType.DMA,
        ],
    )(functools.partial(_gather_body, split))
    return run(uidx, iidx, t_umf, t_imf, t_umlp, t_imlp)


def _dense_body(x, u_mf, i_mf, W1, b1, W2, b2, W3, b3, wo, bo, *rest):
    out = rest[-1]
    f32 = jnp.float32
    bf16 = jnp.bfloat16
    h = jnp.dot(x[...].astype(bf16), W1[...], preferred_element_type=f32)
    h = jnp.maximum(h + b1[...], 0.0).astype(bf16)
    h = jnp.dot(h, W2[...], preferred_element_type=f32)
    h = jnp.maximum(h + b2[...], 0.0).astype(bf16)
    h = jnp.dot(h, W3[...], preferred_element_type=f32)
    h = jnp.maximum(h + b3[...], 0.0)
    mf = u_mf[...] * i_mf[...]
    wo_v = wo[...]
    q = mf * wo_v[:, :_D] + h * wo_v[:, _D:]
    q3 = q.reshape(_BBLK // _D, _D, _D)
    acc = jnp.sum(q3, axis=2)
    out[...] = jax.nn.sigmoid(acc + bo[...])


def _dense_tc(split, x_g, u_mf_g, i_mf_g, W1, b1, W2, b2, W3, b3, wov, bov,
              prev_out):
    off = split * (_BS // _BBLK)
    blk = lambda r, c: pl.BlockSpec((r, c), lambda i: (0, 0))
    act = pl.BlockSpec((_BBLK, _D), lambda i: (i, 0))
    out_spec = pl.BlockSpec((_BBLK // _D, _D), lambda i: (i + off, 0))
    in_specs = [pl.BlockSpec((_BBLK, 2 * _D), lambda i: (i, 0)), act, act,
                blk(256, 1024), blk(1, 1024),
                blk(1024, 512), blk(1, 512),
                blk(512, 128), blk(1, 128),
                blk(1, 256), blk(1, 1)]
    args = [x_g, u_mf_g, i_mf_g, W1, b1, W2, b2, W3, b3, wov, bov]
    aliases = {}
    if prev_out is not None:
        in_specs.append(pl.BlockSpec(memory_space=pl.ANY))
        args.append(prev_out)
        aliases = {11: 0}
    return pl.pallas_call(
        _dense_body,
        grid=(_BS // _BBLK,),
        in_specs=in_specs,
        out_specs=out_spec,
        out_shape=jax.ShapeDtypeStruct((_B // _D, _D), jnp.float32),
        input_output_aliases=aliases,
        compiler_params=pltpu.CompilerParams(
            dimension_semantics=("arbitrary",)),
    )(*args)


def kernel(user_indices, item_indices, emb_user_mf, emb_item_mf,
           emb_user_mlp, emb_item_mlp, W1, b1, W2, b2, W3, b3, W_out, b_out):
    uidx = user_indices.astype(jnp.int32)
    iidx = item_indices.astype(jnp.int32)
    bf16 = jnp.bfloat16
    W1c = W1.astype(bf16)
    W2c = W2.astype(bf16)
    W3c = W3.astype(bf16)
    b1v = b1.reshape(1, -1)
    b2v = b2.reshape(1, -1)
    b3v = b3.reshape(1, -1)
    wov = W_out.reshape(1, -1)
    bov = b_out.reshape(1, 1)
    out = None
    for s in range(_NSPLIT):
        x_g, o_umf, o_imf = _gather_sc(
            s, uidx, iidx, emb_user_mf, emb_item_mf,
            emb_user_mlp, emb_item_mlp)
        out = _dense_tc(s, x_g, o_umf, o_imf,
                        W1c, b1v, W2c, b2v, W3c, b3v, wov, bov, out)
    return out.reshape(_B, 1)
